# baseline (speedup 1.0000x reference)
import functools

import jax
import jax.numpy as jnp
from jax import lax
from jax.experimental import pallas as pl
from jax.experimental.pallas import tpu as pltpu
from jax.experimental.pallas import tpu_sc as plsc

BATCH = 16384
UID_DIM = 64
GEN_DIM = 8
AGE_DIM = 16
OCC_DIM = 16
FINAL = 128
NC, NS = 2, 16
NW = NC * NS
BPW = BATCH // NW


def _sc_gather(sample, user_id_emb, gender_emb, age_emb, occupation_emb,
               map_gender, map_age, map_occupation):
    mesh = plsc.VectorSubcoreMesh(core_axis_name="c", subcore_axis_name="s")

    @functools.partial(
        pl.kernel,
        mesh=mesh,
        compiler_params=pltpu.CompilerParams(use_tc_tiling_on_sc=False),
        out_type=(
            jax.ShapeDtypeStruct((BATCH, UID_DIM), jnp.float32),
            jax.ShapeDtypeStruct((BATCH, GEN_DIM), jnp.float32),
            jax.ShapeDtypeStruct((BATCH, AGE_DIM), jnp.float32),
            jax.ShapeDtypeStruct((BATCH, OCC_DIM), jnp.float32),
        ),
        scratch_types=[
            pltpu.VMEM((BPW,), jnp.int32),
            pltpu.VMEM((BPW,), jnp.int32),
            pltpu.VMEM((BPW,), jnp.int32),
            pltpu.VMEM((BPW,), jnp.int32),
            pltpu.VMEM((BPW, UID_DIM), jnp.float32),
            pltpu.VMEM((BPW, GEN_DIM), jnp.float32),
            pltpu.VMEM((BPW, AGE_DIM), jnp.float32),
            pltpu.VMEM((BPW, OCC_DIM), jnp.float32),
            pltpu.SemaphoreType.DMA,
            pltpu.SemaphoreType.DMA,
        ],
    )
    def k(sample_hbm, uid_hbm, gen_hbm, age_hbm, occ_hbm,
          mg_hbm, ma_hbm, mo_hbm,
          uid_out, gen_out, age_out, occ_out,
          idx_v, gi_v, ai_v, oi_v, uid_v, gen_v, age_v, occ_v,
          sem_u, sem_m):
        wid = lax.axis_index("s") * NC + lax.axis_index("c")
        base = wid * BPW
        pltpu.sync_copy(sample_hbm.at[pl.ds(base, BPW)], idx_v)
        uid_dma = pltpu.async_copy(uid_hbm.at[idx_v], uid_v, sem_u)
        mg_dma = pltpu.async_copy(mg_hbm.at[idx_v], gi_v, sem_m)
        ma_dma = pltpu.async_copy(ma_hbm.at[idx_v], ai_v, sem_m)
        mo_dma = pltpu.async_copy(mo_hbm.at[idx_v], oi_v, sem_m)
        mg_dma.wait()
        pltpu.sync_copy(gen_hbm.at[gi_v], gen_v)
        ma_dma.wait()
        pltpu.sync_copy(age_hbm.at[ai_v], age_v)
        mo_dma.wait()
        pltpu.sync_copy(occ_hbm.at[oi_v], occ_v)
        pltpu.sync_copy(gen_v, gen_out.at[pl.ds(base, BPW)])
        pltpu.sync_copy(age_v, age_out.at[pl.ds(base, BPW)])
        pltpu.sync_copy(occ_v, occ_out.at[pl.ds(base, BPW)])
        uid_dma.wait()
        pltpu.sync_copy(uid_v, uid_out.at[pl.ds(base, BPW)])

    return k(sample, user_id_emb, gender_emb, age_emb, occupation_emb,
             map_gender, map_age, map_occupation)


def _tc_dense(f_uid, f_gen, f_age, f_occ, W, b):
    BLK = 2048

    def body(fu, fg, fa, fo, w_ref, b_ref, o_ref):
        dn = (((1,), (0,)), ((), ()))
        acc = lax.dot_general(fu[...], w_ref[0:UID_DIM, :], dn,
                              preferred_element_type=jnp.float32)
        acc += lax.dot_general(fg[...], w_ref[UID_DIM:UID_DIM + GEN_DIM, :],
                               dn, preferred_element_type=jnp.float32)
        acc += lax.dot_general(fa[...],
                               w_ref[UID_DIM + GEN_DIM:
                                     UID_DIM + GEN_DIM + AGE_DIM, :],
                               dn, preferred_element_type=jnp.float32)
        acc += lax.dot_general(fo[...],
                               w_ref[UID_DIM + GEN_DIM + AGE_DIM:, :],
                               dn, preferred_element_type=jnp.float32)
        o_ref[...] = jnp.tanh(acc + b_ref[...])

    return pl.pallas_call(
        body,
        grid=(BATCH // BLK,),
        in_specs=[
            pl.BlockSpec((BLK, UID_DIM), lambda i: (i, 0)),
            pl.BlockSpec((BLK, GEN_DIM), lambda i: (i, 0)),
            pl.BlockSpec((BLK, AGE_DIM), lambda i: (i, 0)),
            pl.BlockSpec((BLK, OCC_DIM), lambda i: (i, 0)),
            pl.BlockSpec((UID_DIM + GEN_DIM + AGE_DIM + OCC_DIM, FINAL),
                         lambda i: (0, 0)),
            pl.BlockSpec((1, FINAL), lambda i: (0, 0)),
        ],
        out_specs=pl.BlockSpec((BLK, FINAL), lambda i: (i, 0)),
        out_shape=jax.ShapeDtypeStruct((BATCH, FINAL), jnp.float32),
    )(f_uid, f_gen, f_age, f_occ, W, b.reshape(1, FINAL))


def kernel(sample, user_id_emb, gender_emb, age_emb, occupation_emb,
           map_gender, map_age, map_occupation, W, b):
    f_uid, f_gen, f_age, f_occ = _sc_gather(
        sample.astype(jnp.int32), user_id_emb, gender_emb, age_emb,
        occupation_emb, map_gender.astype(jnp.int32),
        map_age.astype(jnp.int32), map_occupation.astype(jnp.int32))
    return _tc_dense(f_uid, f_gen, f_age, f_occ, W, b)


# speedup vs baseline: 1.9600x; 1.9600x over previous
import functools

import jax
import jax.numpy as jnp
from jax import lax
from jax.experimental import pallas as pl
from jax.experimental.pallas import tpu as pltpu
from jax.experimental.pallas import tpu_sc as plsc

BATCH = 16384
UID_NUM = 1000000
UID_DIM = 64
GEN_DIM = 8
AGE_DIM = 16
OCC_DIM = 16
GEN_NUM, AGE_NUM, OCC_NUM = 3, 100, 500
FINAL = 128
NC, NS, L = 2, 16, 16
NW = NC * NS
BPW = BATCH // NW
CH = 128
APACK = 64
BU = 4096
NBLK = 245
PAIR_ROWS = NBLK * (BU // 2)


def _tc_pairize(tabT):
    def body(x_ref, o_ref):
        lo = jnp.transpose(x_ref[:, :BU // 2], (1, 0))
        hi = jnp.transpose(x_ref[:, BU // 2:], (1, 0))
        o_ref[...] = jnp.concatenate([lo, hi], axis=1)

    return pl.pallas_call(
        body,
        grid=(NBLK,),
        in_specs=[pl.BlockSpec((UID_DIM, BU), lambda j: (0, j))],
        out_specs=pl.BlockSpec((BU // 2, FINAL), lambda j: (j, 0)),
        out_shape=jax.ShapeDtypeStruct((PAIR_ROWS, FINAL), jnp.float32),
    )(tabT)


def _sc_pair_gather(sample, pair):
    mesh = plsc.VectorSubcoreMesh(core_axis_name="c", subcore_axis_name="s")

    @functools.partial(
        pl.kernel,
        mesh=mesh,
        compiler_params=pltpu.CompilerParams(use_tc_tiling_on_sc=True,
                                             needs_layout_passes=False),
        out_type=jax.ShapeDtypeStruct((BATCH, FINAL), jnp.float32),
        scratch_types=[
            pltpu.VMEM((BPW,), jnp.int32),
            pltpu.VMEM((BPW,), jnp.int32),
            pltpu.VMEM((CH, FINAL), jnp.float32),
        ],
    )
    def k(sample_hbm, pair_hbm, out_hbm, idx_v, kidx_v, rows_v):
        wid = lax.axis_index("s") * NC + lax.axis_index("c")
        base = wid * BPW
        pltpu.sync_copy(sample_hbm.at[pl.ds(base, BPW)], idx_v)

        @pl.loop(0, BPW // L)
        def _(g):
            s = idx_v[pl.ds(g * L, L)]
            kidx_v[pl.ds(g * L, L)] = (
                lax.shift_left(lax.shift_right_logical(s, 12), 11)
                + (s & (BU // 2 - 1)))

        @pl.loop(0, BPW // CH)
        def _(c):
            cbase = c * CH
            pltpu.sync_copy(pair_hbm.at[kidx_v.at[pl.ds(cbase, CH)]], rows_v)
            pltpu.sync_copy(rows_v, out_hbm.at[pl.ds(base + cbase, CH)])

    return k(sample, pair)


def _sc_attr_pack(sample, gen16, age16, occ16, mg, ma, mo):
    mesh = plsc.VectorSubcoreMesh(core_axis_name="c", subcore_axis_name="s")

    @functools.partial(
        pl.kernel,
        mesh=mesh,
        compiler_params=pltpu.CompilerParams(use_tc_tiling_on_sc=False,
                                             needs_layout_passes=False),
        out_type=jax.ShapeDtypeStruct((BATCH, APACK), jnp.float32),
        scratch_types=[
            pltpu.VMEM((BPW,), jnp.int32),
            pltpu.VMEM((BPW,), jnp.int32),
            pltpu.VMEM((BPW,), jnp.int32),
            pltpu.VMEM((BPW,), jnp.int32),
            pltpu.VMEM((GEN_NUM, L), jnp.float32),
            pltpu.VMEM((AGE_NUM, L), jnp.float32),
            pltpu.VMEM((OCC_NUM, L), jnp.float32),
            pltpu.VMEM((BPW, APACK), jnp.float32),
            pltpu.SemaphoreType.DMA,
        ],
    )
    def k(sample_hbm, gen_hbm, age_hbm, occ_hbm, mg_hbm, ma_hbm, mo_hbm,
          attr_out, idx_v, gi_v, ai_v, oi_v, genv, agev, occv, pack_v, sem):
        wid = lax.axis_index("s") * NC + lax.axis_index("c")
        base = wid * BPW
        pltpu.sync_copy(sample_hbm.at[pl.ds(base, BPW)], idx_v)
        mg_dma = pltpu.async_copy(mg_hbm.at[idx_v], gi_v, sem)
        pltpu.sync_copy(gen_hbm, genv)
        pltpu.sync_copy(age_hbm, agev)
        pltpu.sync_copy(occ_hbm, occv)
        mg_dma.wait()
        ma_dma = pltpu.async_copy(ma_hbm.at[idx_v], ai_v, sem)
        mo_dma = pltpu.async_copy(mo_hbm.at[idx_v], oi_v, sem)
        ma_dma.wait()
        mo_dma.wait()

        lanes = lax.iota(jnp.int32, L)

        @pl.loop(0, BPW // L)
        def _(g):
            j = g * L + lanes
            off = g * L
            gvec = gi_v[pl.ds(off, L)]
            avec = ai_v[pl.ds(off, L)]
            ovec = oi_v[pl.ds(off, L)]
            for w in range(L):
                wv = jnp.full((L,), w, jnp.int32)
                plsc.store_scatter(
                    pack_v, [j, wv],
                    plsc.load_gather(genv, [gvec, wv]))
                plsc.store_scatter(
                    pack_v, [j, wv + L],
                    plsc.load_gather(agev, [avec, wv]))
                plsc.store_scatter(
                    pack_v, [j, wv + 2 * L],
                    plsc.load_gather(occv, [ovec, wv]))

        pltpu.sync_copy(pack_v, attr_out.at[pl.ds(base, BPW)])

    return k(sample, gen16, age16, occ16, mg, ma, mo)


def _tc_dense(pairrows, attr, samp2d, W2, Wa, b):
    BLK = 2048
    KA = 3 * L

    def body(f_ref, a_ref, s_ref, w2_ref, wa_ref, b_ref, o_ref):
        dn = (((1,), (0,)), ((), ()))
        half = lax.shift_right_logical(s_ref[...], 11) & 1
        lane_half = lax.shift_right_logical(
            lax.broadcasted_iota(jnp.int32, (BLK, FINAL), 1), 6)
        sel = lane_half == half
        x = jnp.where(sel, f_ref[...], 0.0)
        acc = lax.dot_general(x, w2_ref[...], dn,
                              preferred_element_type=jnp.float32)
        acc += lax.dot_general(a_ref[:, :KA], wa_ref[...], dn,
                               preferred_element_type=jnp.float32)
        o_ref[...] = jnp.tanh(acc + b_ref[...])

    return pl.pallas_call(
        body,
        grid=(BATCH // BLK,),
        in_specs=[
            pl.BlockSpec((BLK, FINAL), lambda i: (i, 0)),
            pl.BlockSpec((BLK, APACK), lambda i: (i, 0)),
            pl.BlockSpec((BLK, 1), lambda i: (i, 0)),
            pl.BlockSpec((FINAL, FINAL), lambda i: (0, 0)),
            pl.BlockSpec((KA, FINAL), lambda i: (0, 0)),
            pl.BlockSpec((1, FINAL), lambda i: (0, 0)),
        ],
        out_specs=pl.BlockSpec((BLK, FINAL), lambda i: (i, 0)),
        out_shape=jax.ShapeDtypeStruct((BATCH, FINAL), jnp.float32),
    )(pairrows, attr, samp2d, W2, Wa, b.reshape(1, FINAL))


def kernel(sample, user_id_emb, gender_emb, age_emb, occupation_emb,
           map_gender, map_age, map_occupation, W, b):
    sample = sample.astype(jnp.int32)
    tabT = user_id_emb.T
    pair = _tc_pairize(tabT)
    gen16 = jnp.pad(gender_emb, ((0, 0), (0, L - GEN_DIM)))
    zeros8 = jnp.zeros((L - GEN_DIM, FINAL), jnp.float32)
    Wu = W[:UID_DIM]
    W2 = jnp.concatenate([Wu, Wu], axis=0)
    Wa = jnp.concatenate([
        W[UID_DIM:UID_DIM + GEN_DIM], zeros8,
        W[UID_DIM + GEN_DIM:],
    ], axis=0)
    attr = _sc_attr_pack(sample, gen16, age_emb, occupation_emb,
                         map_gender.astype(jnp.int32),
                         map_age.astype(jnp.int32),
                         map_occupation.astype(jnp.int32))
    pairrows = _sc_pair_gather(sample, pair)
    return _tc_dense(pairrows, attr, sample.reshape(BATCH, 1), W2, Wa, b)


# speedup vs baseline: 1.9624x; 1.0012x over previous
import functools

import jax
import jax.numpy as jnp
from jax import lax
from jax.experimental import pallas as pl
from jax.experimental.pallas import tpu as pltpu
from jax.experimental.pallas import tpu_sc as plsc

BATCH = 16384
UID_NUM = 1000000
UID_DIM = 64
GEN_DIM = 8
AGE_DIM = 16
OCC_DIM = 16
GEN_NUM, AGE_NUM, OCC_NUM = 3, 100, 500
FINAL = 128
NC, NS, L = 2, 16, 16
NW = NC * NS
BPW = BATCH // NW
CH = 128
APACK = 64
BU = 4096
NBLK = 245
PAIR_ROWS = NBLK * (BU // 2)


def _tc_pairize(tabT, eye64):
    dn = (((0,), (0,)), ((), ()))

    def body(x_ref, e_ref, o_ref):
        lo = lax.dot_general(x_ref[:, :BU // 2], e_ref[...], dn,
                             preferred_element_type=jnp.float32)
        hi = lax.dot_general(x_ref[:, BU // 2:], e_ref[...], dn,
                             preferred_element_type=jnp.float32)
        o_ref[...] = jnp.concatenate([lo, hi], axis=1)

    return pl.pallas_call(
        body,
        grid=(NBLK,),
        in_specs=[pl.BlockSpec((UID_DIM, BU), lambda j: (0, j)),
                  pl.BlockSpec((UID_DIM, UID_DIM), lambda j: (0, 0))],
        out_specs=pl.BlockSpec((BU // 2, FINAL), lambda j: (j, 0)),
        out_shape=jax.ShapeDtypeStruct((PAIR_ROWS, FINAL), jnp.float32),
    )(tabT, eye64)


def _sc_pair_gather(sample, pair):
    mesh = plsc.VectorSubcoreMesh(core_axis_name="c", subcore_axis_name="s")

    @functools.partial(
        pl.kernel,
        mesh=mesh,
        compiler_params=pltpu.CompilerParams(use_tc_tiling_on_sc=True,
                                             needs_layout_passes=False),
        out_type=jax.ShapeDtypeStruct((BATCH, FINAL), jnp.float32),
        scratch_types=[
            pltpu.VMEM((BPW,), jnp.int32),
            pltpu.VMEM((BPW,), jnp.int32),
            pltpu.VMEM((CH, FINAL), jnp.float32),
        ],
    )
    def k(sample_hbm, pair_hbm, out_hbm, idx_v, kidx_v, rows_v):
        wid = lax.axis_index("s") * NC + lax.axis_index("c")
        base = wid * BPW
        pltpu.sync_copy(sample_hbm.at[pl.ds(base, BPW)], idx_v)

        @pl.loop(0, BPW // L)
        def _(g):
            s = idx_v[pl.ds(g * L, L)]
            kidx_v[pl.ds(g * L, L)] = (
                lax.shift_left(lax.shift_right_logical(s, 12), 11)
                + (s & (BU // 2 - 1)))

        @pl.loop(0, BPW // CH)
        def _(c):
            cbase = c * CH
            pltpu.sync_copy(pair_hbm.at[kidx_v.at[pl.ds(cbase, CH)]], rows_v)
            pltpu.sync_copy(rows_v, out_hbm.at[pl.ds(base + cbase, CH)])

    return k(sample, pair)


def _sc_attr_pack(sample, gen16, age16, occ16, mg, ma, mo):
    mesh = plsc.VectorSubcoreMesh(core_axis_name="c", subcore_axis_name="s")

    @functools.partial(
        pl.kernel,
        mesh=mesh,
        compiler_params=pltpu.CompilerParams(use_tc_tiling_on_sc=False,
                                             needs_layout_passes=False),
        out_type=jax.ShapeDtypeStruct((BATCH, APACK), jnp.float32),
        scratch_types=[
            pltpu.VMEM((BPW,), jnp.int32),
            pltpu.VMEM((BPW,), jnp.int32),
            pltpu.VMEM((BPW,), jnp.int32),
            pltpu.VMEM((BPW,), jnp.int32),
            pltpu.VMEM((GEN_NUM, L), jnp.float32),
            pltpu.VMEM((AGE_NUM, L), jnp.float32),
            pltpu.VMEM((OCC_NUM, L), jnp.float32),
            pltpu.VMEM((BPW, APACK), jnp.float32),
            pltpu.SemaphoreType.DMA,
        ],
    )
    def k(sample_hbm, gen_hbm, age_hbm, occ_hbm, mg_hbm, ma_hbm, mo_hbm,
          attr_out, idx_v, gi_v, ai_v, oi_v, genv, agev, occv, pack_v, sem):
        wid = lax.axis_index("s") * NC + lax.axis_index("c")
        base = wid * BPW
        pltpu.sync_copy(sample_hbm.at[pl.ds(base, BPW)], idx_v)
        mg_dma = pltpu.async_copy(mg_hbm.at[idx_v], gi_v, sem)
        pltpu.sync_copy(gen_hbm, genv)
        pltpu.sync_copy(age_hbm, agev)
        pltpu.sync_copy(occ_hbm, occv)
        mg_dma.wait()
        ma_dma = pltpu.async_copy(ma_hbm.at[idx_v], ai_v, sem)
        mo_dma = pltpu.async_copy(mo_hbm.at[idx_v], oi_v, sem)
        ma_dma.wait()
        mo_dma.wait()

        lanes = lax.iota(jnp.int32, L)

        @pl.loop(0, BPW // L)
        def _(g):
            j = g * L + lanes
            off = g * L
            gvec = gi_v[pl.ds(off, L)]
            avec = ai_v[pl.ds(off, L)]
            ovec = oi_v[pl.ds(off, L)]
            for w in range(L):
                wv = jnp.full((L,), w, jnp.int32)
                plsc.store_scatter(
                    pack_v, [j, wv],
                    plsc.load_gather(genv, [gvec, wv]))
                plsc.store_scatter(
                    pack_v, [j, wv + L],
                    plsc.load_gather(agev, [avec, wv]))
                plsc.store_scatter(
                    pack_v, [j, wv + 2 * L],
                    plsc.load_gather(occv, [ovec, wv]))

        pltpu.sync_copy(pack_v, attr_out.at[pl.ds(base, BPW)])

    return k(sample, gen16, age16, occ16, mg, ma, mo)


def _tc_dense(pairrows, attr, samp2d, W2, Wa, b):
    BLK = 2048
    KA = 3 * L

    def body(f_ref, a_ref, s_ref, w2_ref, wa_ref, b_ref, o_ref):
        dn = (((1,), (0,)), ((), ()))
        half = lax.shift_right_logical(s_ref[...], 11) & 1
        lane_half = lax.shift_right_logical(
            lax.broadcasted_iota(jnp.int32, (BLK, FINAL), 1), 6)
        sel = lane_half == half
        x = jnp.where(sel, f_ref[...], 0.0)
        acc = lax.dot_general(x, w2_ref[...], dn,
                              preferred_element_type=jnp.float32)
        acc += lax.dot_general(a_ref[:, :KA], wa_ref[...], dn,
                               preferred_element_type=jnp.float32)
        o_ref[...] = jnp.tanh(acc + b_ref[...])

    return pl.pallas_call(
        body,
        grid=(BATCH // BLK,),
        in_specs=[
            pl.BlockSpec((BLK, FINAL), lambda i: (i, 0)),
            pl.BlockSpec((BLK, APACK), lambda i: (i, 0)),
            pl.BlockSpec((BLK, 1), lambda i: (i, 0)),
            pl.BlockSpec((FINAL, FINAL), lambda i: (0, 0)),
            pl.BlockSpec((KA, FINAL), lambda i: (0, 0)),
            pl.BlockSpec((1, FINAL), lambda i: (0, 0)),
        ],
        out_specs=pl.BlockSpec((BLK, FINAL), lambda i: (i, 0)),
        out_shape=jax.ShapeDtypeStruct((BATCH, FINAL), jnp.float32),
    )(pairrows, attr, samp2d, W2, Wa, b.reshape(1, FINAL))


def kernel(sample, user_id_emb, gender_emb, age_emb, occupation_emb,
           map_gender, map_age, map_occupation, W, b):
    sample = sample.astype(jnp.int32)
    tabT = user_id_emb.T
    pair = _tc_pairize(tabT, jnp.eye(UID_DIM, dtype=jnp.float32))
    gen16 = jnp.pad(gender_emb, ((0, 0), (0, L - GEN_DIM)))
    zeros8 = jnp.zeros((L - GEN_DIM, FINAL), jnp.float32)
    Wu = W[:UID_DIM]
    W2 = jnp.concatenate([Wu, Wu], axis=0)
    Wa = jnp.concatenate([
        W[UID_DIM:UID_DIM + GEN_DIM], zeros8,
        W[UID_DIM + GEN_DIM:],
    ], axis=0)
    attr = _sc_attr_pack(sample, gen16, age_emb, occupation_emb,
                         map_gender.astype(jnp.int32),
                         map_age.astype(jnp.int32),
                         map_occupation.astype(jnp.int32))
    pairrows = _sc_pair_gather(sample, pair)
    return _tc_dense(pairrows, attr, sample.reshape(BATCH, 1), W2, Wa, b)
